# baseline (speedup 1.0000x reference)
import math

import jax
import jax.numpy as jnp
from jax.experimental import pallas as pl
from jax.experimental.pallas import tpu as pltpu

H = 8
D = 256
DK = D // H
B = 2
N = 2048
M = 4096

TM = 1024
MT = M // TM
NT = N // MT

_INV_DK = 1.0 / DK
_SQRT_DK = math.sqrt(DK)
_LN_EPS = 1e-5
_HODGE_EPS = 1e-6

_HI = jax.lax.Precision.HIGHEST


def _dot(x, y, precision=None):
    return jax.lax.dot_general(x, y, (((1,), (0,)), ((), ())),
                               preferred_element_type=jnp.float32,
                               precision=precision)


def _ln(x, e8, et8, g, b, sum_prec):
    mu = _dot(_dot(x, e8, sum_prec) * _INV_DK, et8, _HI)
    xc = x - mu
    var = _dot(_dot(xc * xc, e8, sum_prec) * _INV_DK, et8, _HI)
    return (xc / jnp.sqrt(var + _LN_EPS)) * g + b


def _body(gb_ref, wq_ref, wk_ref, weq_ref, wek_ref, wo_ref, e8_ref, et8_ref,
          v_ref, e_ref, d0_ref, out_ref, acc_ref, hv_ref):
    mi = pl.program_id(1)
    e8 = e8_ref[...]
    et8 = et8_ref[...]

    @pl.when(mi == 0)
    def _init():
        acc_ref[...] = jnp.zeros_like(acc_ref)

    vs = v_ref[0, pl.ds(mi * NT, NT), :]
    q = jnp.dot(vs, wq_ref[...], preferred_element_type=jnp.float32)
    k = jnp.dot(vs, wk_ref[...], preferred_element_type=jnp.float32)
    q = _ln(q, e8, et8, gb_ref[0:1, :], gb_ref[1:2, :], _HI)
    k = _ln(k, e8, et8, gb_ref[2:3, :], gb_ref[3:4, :], _HI)
    hv8 = _dot(q * k, e8, _HI) / _SQRT_DK
    hv_ref[pl.ds(mi * NT, NT), :] = _dot(1.0 / (hv8 + _HODGE_EPS), et8, _HI)

    eb = e_ref[0]
    eq = jnp.dot(eb, weq_ref[...], preferred_element_type=jnp.float32)
    ek = jnp.dot(eb, wek_ref[...], preferred_element_type=jnp.float32)
    eq = _ln(eq, e8, et8, gb_ref[4:5, :], gb_ref[5:6, :], None)
    ek = _ln(ek, e8, et8, gb_ref[6:7, :], gb_ref[7:8, :], None)
    he = _dot(_dot(eq * ek, e8, None) / _SQRT_DK, et8, _HI)

    d0b = d0_ref[0].astype(jnp.bfloat16)
    x1 = jnp.dot(d0b, v_ref[0].astype(jnp.bfloat16),
                 preferred_element_type=jnp.float32)
    x1 = x1 * he
    acc_ref[...] += jax.lax.dot_general(
        d0b, x1.astype(jnp.bfloat16), (((0,), (0,)), ((), ())),
        preferred_element_type=jnp.float32)

    @pl.when(mi == MT - 1)
    def _fin():
        out_ref[0] = jnp.dot(acc_ref[...] * hv_ref[...], wo_ref[...],
                             preferred_element_type=jnp.float32)


def kernel(v, e, d_0, v_idx, e_idx, W_vQ, W_vK, W_vO, W_eQ, W_eK,
           g_vq, b_vq, g_vk, b_vk, g_eq, b_eq, g_ek, b_ek):
    del v_idx, e_idx
    f32 = jnp.float32
    idx = jnp.arange(D)
    e8 = (idx[:, None] // DK == jnp.arange(H)[None, :]).astype(f32)
    et8 = e8.T

    wq = W_vQ.T
    wk = W_vK.T
    weq = W_eQ.T
    wek = W_eK.T
    wo = W_vO.T
    gb = jnp.concatenate([
        g_vq.reshape(1, D), b_vq.reshape(1, D),
        g_vk.reshape(1, D), b_vk.reshape(1, D),
        g_eq.reshape(1, D), b_eq.reshape(1, D),
        g_ek.reshape(1, D), b_ek.reshape(1, D)], axis=0)

    full = lambda shape: pl.BlockSpec(shape, lambda b_, m_: (0,) * len(shape))
    out = pl.pallas_call(
        _body,
        grid=(B, MT),
        in_specs=[
            full((8, D)),
            full((D, D)),
            full((D, D)),
            full((D, D)),
            full((D, D)),
            full((D, D)),
            full((D, H)),
            full((H, D)),
            pl.BlockSpec((1, N, D), lambda b_, m_: (b_, 0, 0)),
            pl.BlockSpec((1, TM, D), lambda b_, m_: (b_, m_, 0)),
            pl.BlockSpec((1, TM, N), lambda b_, m_: (b_, m_, 0)),
        ],
        out_specs=pl.BlockSpec((1, N, D), lambda b_, m_: (b_, 0, 0)),
        out_shape=jax.ShapeDtypeStruct((B, N, D), f32),
        scratch_shapes=[
            pltpu.VMEM((N, D), f32),
            pltpu.VMEM((N, D), f32),
        ],
        compiler_params=pltpu.CompilerParams(
            dimension_semantics=("arbitrary", "arbitrary"),
        ),
    )(gb, wq, wk, weq, wek, wo, e8, et8, v, e, d_0)
    return out
